# baseline (speedup 1.0000x reference)
import functools

import jax
import jax.numpy as jnp
from jax import lax
from jax.experimental import pallas as pl
from jax.experimental.pallas import tpu as pltpu
from jax.experimental.pallas import tpu_sc as plsc

N_NODES = 10000
N_EDGES = 320000
HIDDEN = 256

NC = 2
NS = 16
NW = NC * NS
L = 16

NPAD = 10240
EPT = N_EDGES // NW
SLOT = NPAD // NS
UE = 5
UZ = 8

R_TC = 2048
NB = NPAD // R_TC

_mesh = plsc.VectorSubcoreMesh(core_axis_name="c", subcore_axis_name="s")
_sc_params = pltpu.CompilerParams(
    needs_layout_passes=False, use_tc_tiling_on_sc=False)


def _rsqrt16(x):
    i = plsc.bitcast(x, jnp.int32)
    i = jnp.int32(0x5F3759DF) - (i >> 1)
    y = plsc.bitcast(i, jnp.float32)
    for _ in range(3):
        y = y * (1.5 - 0.5 * x * y * y)
    return y


@functools.partial(
    pl.kernel,
    compiler_params=_sc_params,
    out_type=(
        jax.ShapeDtypeStruct((NC, NPAD), jnp.float32),
        jax.ShapeDtypeStruct((1, NPAD), jnp.float32),
        jax.ShapeDtypeStruct((1, NPAD), jnp.float32),
    ),
    mesh=_mesh,
    scratch_types=[
        pltpu.VMEM((2 * EPT,), jnp.int32),
        pltpu.VMEM((EPT,), jnp.int32),
        pltpu.VMEM((NPAD,), jnp.float32),
        pltpu.VMEM((NPAD,), jnp.float32),
        pltpu.VMEM((NS, SLOT), jnp.float32),
        pltpu.VMEM((SLOT,), jnp.float32),
        pltpu.VMEM((SLOT,), jnp.float32),
        pltpu.VMEM((SLOT,), jnp.float32),
        pltpu.VMEM_SHARED((NS, NPAD), jnp.float32),
        pltpu.VMEM_SHARED((NPAD,), jnp.float32),
    ],
)
def _sc_kernel(ei_hbm, s_hbm, g_hbm, dinv_hbm, t_hbm,
               idx_d, srcv, acc, tfull, red, schunk, dchunk, tchunk,
               slots, t_sh):
    c = lax.axis_index("c")
    s = lax.axis_index("s")
    zeros = jnp.zeros((L,), jnp.float32)
    ones = jnp.ones((L,), jnp.float32)

    def zero_acc():
        def zbody(j, _):
            for u in range(UZ):
                acc[pl.ds((j * UZ + u) * L, L)] = zeros
            return 0

        lax.fori_loop(0, NPAD // (L * UZ), zbody, 0)

    zero_acc()
    pltpu.sync_copy(ei_hbm.at[1, pl.ds(s * EPT, EPT)], idx_d.at[pl.ds(0, EPT)])
    pltpu.sync_copy(ei_hbm.at[1, pl.ds((s + NS) * EPT, EPT)],
                    idx_d.at[pl.ds(EPT, EPT)])

    def h1body(i, _):
        sls = [pl.ds((i * (2 * UE) + u) * L, L) for u in range(2 * UE)]
        iv = [idx_d[sl] for sl in sls]
        for v in iv:
            plsc.addupdate_scatter(acc, [v], ones)
        return 0

    lax.fori_loop(0, 2 * EPT // (L * 2 * UE), h1body, 0)

    pltpu.sync_copy(acc, slots.at[s])
    plsc.subcore_barrier()
    base = s * SLOT
    pltpu.sync_copy(slots.at[:, pl.ds(base, SLOT)], red)
    pltpu.sync_copy(s_hbm.at[pl.ds(base, SLOT)], schunk)

    def nbody(j, _):
        sl = pl.ds(j * L, L)
        deg = jnp.ones((L,), jnp.float32)
        for k in range(NS):
            deg = deg + red[k, sl]
        dinv = _rsqrt16(deg)
        dchunk[sl] = dinv
        tchunk[sl] = dinv * schunk[sl]
        return 0

    lax.fori_loop(0, SLOT // L, nbody, 0)
    pltpu.sync_copy(tchunk, t_sh.at[pl.ds(base, SLOT)])

    @pl.when(c == 0)
    def _():
        pltpu.sync_copy(dchunk, dinv_hbm.at[0, pl.ds(base, SLOT)])
        pltpu.sync_copy(tchunk, t_hbm.at[0, pl.ds(base, SLOT)])

    plsc.subcore_barrier()
    pltpu.sync_copy(t_sh, tfull)

    zero_acc()
    w = c * NS + s
    pltpu.sync_copy(ei_hbm.at[0, pl.ds(w * EPT, EPT)], srcv)
    ec = c * EPT

    def ebody(i, _):
        sls = [pl.ds((i * UE + u) * L, L) for u in range(UE)]
        sv = [srcv[sl] for sl in sls]
        dv = [idx_d[pl.ds(ec + (i * UE + u) * L, L)] for u in range(UE)]
        gv = [plsc.load_gather(tfull, [x]) for x in sv]
        for u in range(UE):
            plsc.addupdate_scatter(acc, [dv[u]], gv[u])
        return 0

    lax.fori_loop(0, EPT // (L * UE), ebody, 0)

    pltpu.sync_copy(acc, slots.at[s])
    plsc.subcore_barrier()
    pltpu.sync_copy(slots.at[:, pl.ds(base, SLOT)], red)

    def gbody(j, _):
        sl = pl.ds(j * L, L)
        tot = jnp.zeros((L,), jnp.float32)
        for k in range(NS):
            tot = tot + red[k, sl]
        tchunk[sl] = tot
        return 0

    lax.fori_loop(0, SLOT // L, gbody, 0)
    pltpu.sync_copy(tchunk, g_hbm.at[c, pl.ds(base, SLOT)])


def _proj_body(g_ref, t_ref, dinv_ref, w1_ref, b1_ref, w2t_ref, b2_ref,
               x_ref):
    g = jnp.sum(g_ref[...], axis=0, keepdims=True)
    y = dinv_ref[...] * (g + t_ref[...])
    outT = (w1_ref[...] * y + b1_ref[...]).astype(jnp.bfloat16)
    xT = jnp.dot(w2t_ref[...], outT,
                 preferred_element_type=jnp.float32)
    x_ref[...] = xT + b2_ref[...]


_proj_tc = pl.pallas_call(
    _proj_body,
    grid=(NB,),
    in_specs=[
        pl.BlockSpec((NC, R_TC), lambda i: (0, i)),
        pl.BlockSpec((1, R_TC), lambda i: (0, i)),
        pl.BlockSpec((1, R_TC), lambda i: (0, i)),
        pl.BlockSpec((HIDDEN, 1), lambda i: (0, 0)),
        pl.BlockSpec((HIDDEN, 1), lambda i: (0, 0)),
        pl.BlockSpec((1, HIDDEN), lambda i: (0, 0)),
        pl.BlockSpec((1, 1), lambda i: (0, 0)),
    ],
    out_specs=pl.BlockSpec((1, R_TC), lambda i: (0, i)),
    out_shape=jax.ShapeDtypeStruct((1, NPAD), jnp.float32),
)


def kernel(state, edge_index, edge_attr, W1, b1, W2, b2):
    s_pad = jnp.pad(state, (0, NPAD - N_NODES))
    g, dinv, t = _sc_kernel(edge_index, s_pad)
    x2 = _proj_tc(g, t, dinv,
                  W1.reshape(HIDDEN, 1), b1.reshape(HIDDEN, 1),
                  W2.reshape(1, HIDDEN).astype(jnp.bfloat16),
                  b2.reshape(1, 1))
    return x2.reshape(NPAD)[:N_NODES].reshape(N_NODES, 1)


# speedup vs baseline: 1.1276x; 1.1276x over previous
import functools

import jax
import jax.numpy as jnp
from jax import lax
from jax.experimental import pallas as pl
from jax.experimental.pallas import tpu as pltpu
from jax.experimental.pallas import tpu_sc as plsc

N_NODES = 10000
N_EDGES = 320000
HIDDEN = 256

NC = 2
NS = 16
NW = NC * NS
L = 16

NPAD = 10240
EPT = N_EDGES // NW
SLOT = NPAD // NS
STAIL = N_NODES - (NS - 1) * SLOT
UE = 5
UZ = 8

R_TC = 2048
NB = NPAD // R_TC

_mesh = plsc.VectorSubcoreMesh(core_axis_name="c", subcore_axis_name="s")
_sc_params = pltpu.CompilerParams(
    needs_layout_passes=False, use_tc_tiling_on_sc=False)


def _rsqrt16(x):
    i = plsc.bitcast(x, jnp.int32)
    i = jnp.int32(0x5F3759DF) - (i >> 1)
    y = plsc.bitcast(i, jnp.float32)
    for _ in range(3):
        y = y * (1.5 - 0.5 * x * y * y)
    return y


@functools.partial(
    pl.kernel,
    compiler_params=_sc_params,
    out_type=(
        jax.ShapeDtypeStruct((NPAD,), jnp.float32),
        jax.ShapeDtypeStruct((NPAD,), jnp.float32),
        jax.ShapeDtypeStruct((NPAD,), jnp.float32),
        jax.ShapeDtypeStruct((NPAD,), jnp.float32),
    ),
    mesh=_mesh,
    scratch_types=[
        pltpu.VMEM((2 * EPT,), jnp.int32),
        pltpu.VMEM((EPT,), jnp.int32),
        pltpu.VMEM((NPAD,), jnp.float32),
        pltpu.VMEM((NPAD,), jnp.float32),
        pltpu.VMEM((NS, SLOT), jnp.float32),
        pltpu.VMEM((SLOT,), jnp.float32),
        pltpu.VMEM((SLOT,), jnp.float32),
        pltpu.VMEM((SLOT,), jnp.float32),
        pltpu.VMEM_SHARED((NS, NPAD), jnp.float32),
        pltpu.VMEM_SHARED((NPAD,), jnp.float32),
        pltpu.SemaphoreType.DMA,
        pltpu.SemaphoreType.DMA,
        pltpu.SemaphoreType.DMA,
    ],
)
def _sc_kernel(ei_hbm, s_hbm, g0_hbm, g1_hbm, dinv_hbm, t_hbm,
               idx_d, srcv, acc, tfull, red, schunk, dchunk, tchunk,
               slots, t_sh, sem_a, sem_b, sem_c):
    c = lax.axis_index("c")
    s = lax.axis_index("s")
    base = s * SLOT
    w = c * NS + s
    zeros = jnp.zeros((L,), jnp.float32)
    ones = jnp.ones((L,), jnp.float32)

    def zero_acc():
        def zbody(j, _):
            for u in range(UZ):
                acc[pl.ds((j * UZ + u) * L, L)] = zeros
            return 0

        lax.fori_loop(0, NPAD // (L * UZ), zbody, 0)

    cp_d1 = pltpu.async_copy(ei_hbm.at[1, pl.ds(s * EPT, EPT)],
                             idx_d.at[pl.ds(0, EPT)], sem_a)
    cp_d2 = pltpu.async_copy(ei_hbm.at[1, pl.ds((s + NS) * EPT, EPT)],
                             idx_d.at[pl.ds(EPT, EPT)], sem_a)
    cp_src = pltpu.async_copy(ei_hbm.at[0, pl.ds(w * EPT, EPT)], srcv, sem_b)

    @pl.when(s < NS - 1)
    def _():
        pltpu.sync_copy(s_hbm.at[pl.ds(base, SLOT)], schunk)

    @pl.when(s == NS - 1)
    def _():
        for j in range(STAIL // L, SLOT // L):
            schunk[pl.ds(j * L, L)] = zeros
        pltpu.sync_copy(s_hbm.at[pl.ds(base, STAIL)],
                        schunk.at[pl.ds(0, STAIL)])

    zero_acc()
    cp_d1.wait()
    cp_d2.wait()

    def h1body(i, _):
        sls = [pl.ds((i * (2 * UE) + u) * L, L) for u in range(2 * UE)]
        iv = [idx_d[sl] for sl in sls]
        for v in iv:
            plsc.addupdate_scatter(acc, [v], ones)
        return 0

    lax.fori_loop(0, 2 * EPT // (L * 2 * UE), h1body, 0)

    pltpu.sync_copy(acc, slots.at[s])
    plsc.subcore_barrier()
    pltpu.sync_copy(slots.at[:, pl.ds(base, SLOT)], red)

    def nbody(j, _):
        sl = pl.ds(j * L, L)
        deg = jnp.ones((L,), jnp.float32)
        for k in range(NS):
            deg = deg + red[k, sl]
        dinv = _rsqrt16(deg)
        dchunk[sl] = dinv
        tchunk[sl] = dinv * schunk[sl]
        return 0

    lax.fori_loop(0, SLOT // L, nbody, 0)
    pltpu.sync_copy(tchunk, t_sh.at[pl.ds(base, SLOT)])

    @pl.when(c == 0)
    def _():
        pltpu.sync_copy(dchunk, dinv_hbm.at[pl.ds(base, SLOT)])
        pltpu.sync_copy(tchunk, t_hbm.at[pl.ds(base, SLOT)])

    plsc.subcore_barrier()

    cp_t = pltpu.async_copy(t_sh, tfull, sem_c)
    zero_acc()
    cp_t.wait()
    cp_src.wait()
    ec = c * EPT

    def ebody(i, _):
        sls = [pl.ds((i * UE + u) * L, L) for u in range(UE)]
        sv = [srcv[sl] for sl in sls]
        dv = [idx_d[pl.ds(ec + (i * UE + u) * L, L)] for u in range(UE)]
        gv = [plsc.load_gather(tfull, [x]) for x in sv]
        for u in range(UE):
            plsc.addupdate_scatter(acc, [dv[u]], gv[u])
        return 0

    lax.fori_loop(0, EPT // (L * UE), ebody, 0)

    pltpu.sync_copy(acc, slots.at[s])
    plsc.subcore_barrier()
    pltpu.sync_copy(slots.at[:, pl.ds(base, SLOT)], red)

    def gbody(j, _):
        sl = pl.ds(j * L, L)
        tot = jnp.zeros((L,), jnp.float32)
        for k in range(NS):
            tot = tot + red[k, sl]
        tchunk[sl] = tot
        return 0

    lax.fori_loop(0, SLOT // L, gbody, 0)

    @pl.when(c == 0)
    def _():
        pltpu.sync_copy(tchunk, g0_hbm.at[pl.ds(base, SLOT)])

    @pl.when(c == 1)
    def _():
        pltpu.sync_copy(tchunk, g1_hbm.at[pl.ds(base, SLOT)])


def _proj_body(g0_ref, g1_ref, t_ref, dinv_ref, w1_ref, b1_ref, w2t_ref,
               b2_ref, x_ref):
    y = dinv_ref[...] * (g0_ref[...] + g1_ref[...] + t_ref[...])
    outT = (w1_ref[...] * y + b1_ref[...]).astype(jnp.bfloat16)
    xT = jnp.dot(w2t_ref[...], outT,
                 preferred_element_type=jnp.float32)
    x_ref[...] = xT + b2_ref[...]


_proj_tc = pl.pallas_call(
    _proj_body,
    grid=(NB,),
    in_specs=[
        pl.BlockSpec((R_TC,), lambda i: (i,)),
        pl.BlockSpec((R_TC,), lambda i: (i,)),
        pl.BlockSpec((R_TC,), lambda i: (i,)),
        pl.BlockSpec((R_TC,), lambda i: (i,)),
        pl.BlockSpec((HIDDEN, 1), lambda i: (0, 0)),
        pl.BlockSpec((HIDDEN, 1), lambda i: (0, 0)),
        pl.BlockSpec((1, HIDDEN), lambda i: (0, 0)),
        pl.BlockSpec((1, 1), lambda i: (0, 0)),
    ],
    out_specs=pl.BlockSpec((1, R_TC), lambda i: (0, i)),
    out_shape=jax.ShapeDtypeStruct((1, NPAD), jnp.float32),
)


def kernel(state, edge_index, edge_attr, W1, b1, W2, b2):
    g0, g1, dinv, t = _sc_kernel(edge_index, state)
    x2 = _proj_tc(g0, g1, t, dinv,
                  W1.reshape(HIDDEN, 1), b1.reshape(HIDDEN, 1),
                  W2.reshape(1, HIDDEN).astype(jnp.bfloat16),
                  b2.reshape(1, 1))
    return x2.reshape(NPAD)[:N_NODES].reshape(N_NODES, 1)
